# baseline (speedup 1.0000x reference)
import jax
import jax.numpy as jnp
from jax.experimental import pallas as pl
from jax.experimental.pallas import tpu as pltpu


def _make_body(b, h, w, d):
    def _body(col_hbm, row_hbm, o_ref, colv, rowv, scratch, insems, sems):
        fetch_col = pltpu.make_async_copy(
            col_hbm.at[pl.ds(0, w)], colv, insems.at[0]
        )
        fetch_row = pltpu.make_async_copy(
            row_hbm.at[pl.ds(0, h)], rowv, insems.at[1]
        )
        fetch_col.start()
        fetch_row.start()
        fetch_col.wait()
        scratch[:, :, :, :d] = jnp.broadcast_to(
            colv[...][None, None, :, :], (2, h, w, d)
        )
        fetch_row.wait()
        scratch[:, :, :, d:] = jnp.broadcast_to(
            rowv[...][None, :, None, :], (2, h, w, d)
        )
        copies = [
            pltpu.make_async_copy(scratch.at[i % 2], o_ref.at[i], sems.at[i])
            for i in range(b)
        ]
        for c in copies:
            c.start()
        for c in copies:
            c.wait()

    return _body


def kernel(x, mask, row_embed, col_embed):
    b = x.shape[0]
    h, w = x.shape[-2], x.shape[-1]
    d = col_embed.shape[-1]
    out_nat = pl.pallas_call(
        _make_body(b, h, w, d),
        grid=(1,),
        in_specs=[
            pl.BlockSpec(memory_space=pl.ANY),
            pl.BlockSpec(memory_space=pl.ANY),
        ],
        out_specs=pl.BlockSpec(memory_space=pl.ANY),
        out_shape=jax.ShapeDtypeStruct((b, h, w, 2 * d), jnp.float32),
        scratch_shapes=[
            pltpu.VMEM((w, d), jnp.float32),
            pltpu.VMEM((h, d), jnp.float32),
            pltpu.VMEM((2, h, w, 2 * d), jnp.float32),
            pltpu.SemaphoreType.DMA((2,)),
            pltpu.SemaphoreType.DMA((b,)),
        ],
    )(col_embed, row_embed)
    return jnp.transpose(out_nat, (0, 3, 1, 2))


# speedup vs baseline: 1.0510x; 1.0510x over previous
import jax
import jax.numpy as jnp
from jax.experimental import pallas as pl
from jax.experimental.pallas import tpu as pltpu


def _make_body(b, h, w, d):
    def _body(col_hbm, row_hbm, o_ref, colv, rowv, scratch, insems, sems):
        fetch_col = pltpu.make_async_copy(
            col_hbm.at[pl.ds(0, w)], colv, insems.at[0]
        )
        fetch_row = pltpu.make_async_copy(
            row_hbm.at[pl.ds(0, h)], rowv, insems.at[1]
        )
        fetch_col.start()
        fetch_row.start()
        fetch_col.wait()
        scratch[:, :, :d] = jnp.broadcast_to(colv[...][None, :, :], (h, w, d))
        fetch_row.wait()
        scratch[:, :, d:] = jnp.broadcast_to(rowv[...][:, None, :], (h, w, d))
        copies = [
            pltpu.make_async_copy(scratch, o_ref.at[i], sems.at[i])
            for i in range(b)
        ]
        for c in copies:
            c.start()
        for c in copies:
            c.wait()

    return _body


def kernel(x, mask, row_embed, col_embed):
    b = x.shape[0]
    h, w = x.shape[-2], x.shape[-1]
    d = col_embed.shape[-1]
    out_nat = pl.pallas_call(
        _make_body(b, h, w, d),
        grid=(1,),
        in_specs=[
            pl.BlockSpec(memory_space=pl.ANY),
            pl.BlockSpec(memory_space=pl.ANY),
        ],
        out_specs=pl.BlockSpec(memory_space=pl.ANY),
        out_shape=jax.ShapeDtypeStruct((b, h, w, 2 * d), jnp.float32),
        scratch_shapes=[
            pltpu.VMEM((w, d), jnp.float32),
            pltpu.VMEM((h, d), jnp.float32),
            pltpu.VMEM((h, w, 2 * d), jnp.float32),
            pltpu.SemaphoreType.DMA((2,)),
            pltpu.SemaphoreType.DMA((b,)),
        ],
    )(col_embed, row_embed)
    return jnp.transpose(out_nat, (0, 3, 1, 2))
